# baseline (speedup 1.0000x reference)
import dataclasses
import functools

import jax
import jax.numpy as jnp
from jax import lax
from jax.experimental import pallas as pl
from jax.experimental.pallas import tpu as pltpu
from jax.experimental.pallas import tpu_sc as plsc

N = 10000
E = 320000
D = 128
LANES = 16
NCORES = 2
NSUB = 16
NW = NCORES * NSUB
EPW = E // NW
RB = 80
NB = EPW // RB
ROWS_PER_SUB = N // NSUB
ZROWS = 125


def _sc_compiler_params():
    cp = pltpu.CompilerParams()
    if "needs_layout_passes" in pltpu.CompilerParams.__dataclass_fields__:
        cp = dataclasses.replace(cp, needs_layout_passes=False)
    return cp


def _tc_prep_kernel(x_ref, w_ref, b_ref, wl_ref, wr_ref,
                    feat_ref, sa_ref, at_ref, m_ref):
    feat = lax.dot_general(x_ref[...], w_ref[...], (((1,), (1,)), ((), ())),
                           preferred_element_type=jnp.float32)
    feat = feat + b_ref[...]
    feat_ref[...] = feat
    sa = lax.dot_general(feat, wl_ref[...], (((1,), (0,)), ((), ())),
                         preferred_element_type=jnp.float32)
    at = lax.dot_general(feat, wr_ref[...], (((1,), (0,)), ((), ())),
                         preferred_element_type=jnp.float32)
    sa_ref[...] = sa
    at_ref[...] = at
    m = jnp.max(sa) + jnp.max(at)
    m_ref[...] = jnp.where(m > 0, m, 0.01 * m)[None, None]


def _tc_recip_kernel(d_ref, r_ref):
    d = d_ref[0:1, :] + d_ref[1:2, :]
    r_ref[...] = 1.0 / jnp.maximum(d, 1e-30)


def _tc_final_kernel(acc_ref, h_ref, phis_ref):
    h = jnp.maximum(acc_ref[0] + acc_ref[1], 0.0)
    h_ref[...] = h
    phis_ref[...] = jnp.sum(h, axis=0, keepdims=True)


def _sc_edge_softmax(selfa, attn, m16, src3, dst3):
    mesh = plsc.VectorSubcoreMesh(core_axis_name="c", subcore_axis_name="s")

    @functools.partial(
        pl.kernel,
        out_type=(jax.ShapeDtypeStruct((NW, NB, RB), jnp.float32),
                  jax.ShapeDtypeStruct((NCORES, N), jnp.float32)),
        mesh=mesh,
        compiler_params=_sc_compiler_params(),
        scratch_types=[
            pltpu.VMEM((N,), jnp.float32),
            pltpu.VMEM((N,), jnp.float32),
            pltpu.VMEM((LANES,), jnp.float32),
            pltpu.VMEM((NB, RB), jnp.int32),
            pltpu.VMEM((NB, RB), jnp.int32),
            pltpu.VMEM((NB, RB), jnp.float32),
            pltpu.VMEM((2000,), jnp.float32),
            pltpu.VMEM_SHARED((N,), jnp.float32),
        ],
    )
    def kern(selfa_hbm, attn_hbm, m_hbm, src_hbm, dst_hbm,
             ex_hbm, denomp_hbm,
             selfa_v, attn_v, m_v, src_v, dst_v, ex_v, zbuf, denom_sh):
        cid = lax.axis_index("c")
        sid = lax.axis_index("s")
        wid = sid * NCORES + cid

        pltpu.sync_copy(selfa_hbm, selfa_v)
        pltpu.sync_copy(attn_hbm, attn_v)
        pltpu.sync_copy(m_hbm, m_v)
        pltpu.sync_copy(src_hbm.at[wid], src_v)
        pltpu.sync_copy(dst_hbm.at[wid], dst_v)

        @pl.when(sid == 0)
        def _():
            @pl.loop(0, 2000 // LANES)
            def _(i):
                zbuf[pl.ds(i * LANES, LANES)] = jnp.zeros((LANES,), jnp.float32)

            @pl.loop(0, N // 2000)
            def _(t):
                pltpu.sync_copy(zbuf, denom_sh.at[pl.ds(t * 2000, 2000)])

        plsc.subcore_barrier()

        mvec = m_v[...]

        @plsc.parallel_loop(0, NB, unroll=4)
        def _(j):
            for k in range(RB // LANES):
                sl = pl.ds(k * LANES, LANES)
                a = (plsc.load_gather(selfa_v, [dst_v.at[j][sl]])
                     + plsc.load_gather(attn_v, [src_v.at[j][sl]]))
                a = jnp.where(a > 0, a, a * 0.01)
                ex_v.at[j][sl] = jnp.exp(a - mvec)

        @pl.loop(0, NB)
        def _(j):
            pltpu.sync_copy(ex_v.at[j], denom_sh.at[dst_v.at[j]], add=True)

        pltpu.sync_copy(ex_v, ex_hbm.at[wid])
        plsc.subcore_barrier()

        @pl.when(sid == 0)
        def _():
            pltpu.sync_copy(denom_sh, denomp_hbm.at[cid])

    return kern(selfa, attn, m16, src3, dst3)


def _sc_aggregate(feat, ex3, recip, src3, dst3):
    mesh = plsc.VectorSubcoreMesh(core_axis_name="c", subcore_axis_name="s")

    CB = 8
    NCHUNK = NB // CB

    @functools.partial(
        pl.kernel,
        out_type=jax.ShapeDtypeStruct((NCORES, N, D), jnp.float32),
        mesh=mesh,
        compiler_params=_sc_compiler_params(),
        scratch_types=[
            pltpu.VMEM((N,), jnp.float32),
            pltpu.VMEM((CB, RB), jnp.int32),
            pltpu.VMEM((CB, RB), jnp.int32),
            pltpu.VMEM((CB, RB), jnp.float32),
            pltpu.VMEM((RB, D), jnp.float32),
            pltpu.VMEM((RB, D), jnp.float32),
            pltpu.SemaphoreType.DMA,
            pltpu.SemaphoreType.DMA,
            pltpu.VMEM_SHARED((N, D), jnp.float32),
        ],
    )
    def kern(feat_hbm, ex_hbm, recip_hbm, src_hbm, dst_hbm, acc_hbm,
             recip_v, src_c, dst_c, e_c, rows0_v, rows1_v, gsem, ssem,
             acc_sh):
        cid = lax.axis_index("c")
        sid = lax.axis_index("s")
        wid = sid * NCORES + cid

        pltpu.sync_copy(recip_hbm, recip_v)

        @pl.loop(0, RB)
        def _(r):
            for c in range(D // LANES):
                rows0_v.at[r][pl.ds(c * LANES, LANES)] = jnp.zeros(
                    (LANES,), jnp.float32)

        for t in range(7):
            pltpu.sync_copy(
                rows0_v, acc_sh.at[pl.ds(sid * ROWS_PER_SUB + t * RB, RB)])
        pltpu.sync_copy(rows0_v.at[pl.ds(0, 65)],
                        acc_sh.at[pl.ds(sid * ROWS_PER_SUB + 560, 65)])
        plsc.subcore_barrier()

        bufs = (rows0_v, rows1_v)

        def scale_block(j2, rows_v):
            for k in range(RB // LANES):
                sl = pl.ds(k * LANES, LANES)
                r16 = plsc.load_gather(recip_v, [dst_c.at[j2][sl]])
                e_c.at[j2][sl] = e_c.at[j2][sl] * r16

            @plsc.parallel_loop(0, RB, unroll=8)
            def _(r):
                esp = plsc.load_gather(
                    e_c, [jnp.full((LANES,), j2, jnp.int32),
                          jnp.full((LANES,), r, jnp.int32)])
                for c in range(D // LANES):
                    sl = pl.ds(c * LANES, LANES)
                    rows_v.at[r][sl] = rows_v.at[r][sl] * esp

        def do_chunk(base, nblk):
            pltpu.sync_copy(src_hbm.at[wid].at[pl.ds(base, nblk)],
                            src_c.at[pl.ds(0, nblk)])
            pltpu.sync_copy(dst_hbm.at[wid].at[pl.ds(base, nblk)],
                            dst_c.at[pl.ds(0, nblk)])
            pltpu.sync_copy(ex_hbm.at[wid].at[pl.ds(base, nblk)],
                            e_c.at[pl.ds(0, nblk)])

            h = pltpu.async_copy(feat_hbm.at[src_c.at[0]], bufs[0], gsem)
            sc = [None, None]
            for j2 in range(nblk):
                h.wait()
                if j2 + 1 < nblk:
                    if sc[(j2 + 1) % 2] is not None:
                        sc[(j2 + 1) % 2].wait()
                        sc[(j2 + 1) % 2] = None
                    h = pltpu.async_copy(feat_hbm.at[src_c.at[j2 + 1]],
                                         bufs[(j2 + 1) % 2], gsem)
                scale_block(j2, bufs[j2 % 2])
                sc[j2 % 2] = pltpu.async_copy(
                    bufs[j2 % 2], acc_sh.at[dst_c.at[j2]], ssem, add=True)
            for b in range(2):
                if sc[b] is not None:
                    sc[b].wait()

        @pl.loop(0, NCHUNK)
        def _(jj):
            do_chunk(jj * CB, CB)

        do_chunk(NCHUNK * CB, NB - NCHUNK * CB)

        plsc.subcore_barrier()
        pltpu.sync_copy(acc_sh.at[pl.ds(sid * 624, 624)],
                        acc_hbm.at[cid].at[pl.ds(sid * 624, 624)])

        @pl.when(sid == 0)
        def _():
            pltpu.sync_copy(acc_sh.at[pl.ds(9984, 16)],
                            acc_hbm.at[cid].at[pl.ds(9984, 16)])

    return kern(feat, ex3, recip, src3, dst3)


def kernel(x, edge_index, W_fc, b_fc, w_l, w_r):
    feat, sa, at, m11 = pl.pallas_call(
        _tc_prep_kernel,
        out_shape=[
            jax.ShapeDtypeStruct((N, D), jnp.float32),
            jax.ShapeDtypeStruct((N, 1), jnp.float32),
            jax.ShapeDtypeStruct((N, 1), jnp.float32),
            jax.ShapeDtypeStruct((1, 1), jnp.float32),
        ],
    )(x, W_fc, b_fc.reshape(1, D), w_l.reshape(D, 1), w_r.reshape(D, 1))

    m16 = jnp.broadcast_to(m11.reshape(()), (LANES,))
    src3 = edge_index[0].reshape(NW, NB, RB)
    dst3 = edge_index[1].reshape(NW, NB, RB)

    ex3, denomp = _sc_edge_softmax(sa.reshape(N), at.reshape(N), m16,
                                   src3, dst3)
    recip = pl.pallas_call(
        _tc_recip_kernel,
        out_shape=jax.ShapeDtypeStruct((1, N), jnp.float32),
    )(denomp)
    accp = _sc_aggregate(feat, ex3, recip.reshape(N), src3, dst3)

    h, phis = pl.pallas_call(
        _tc_final_kernel,
        out_shape=[
            jax.ShapeDtypeStruct((N, D), jnp.float32),
            jax.ShapeDtypeStruct((1, D), jnp.float32),
        ],
    )(accp)
    return h, phis.reshape(D)


# speedup vs baseline: 1.0442x; 1.0442x over previous
import dataclasses
import functools

import jax
import jax.numpy as jnp
from jax import lax
from jax.experimental import pallas as pl
from jax.experimental.pallas import tpu as pltpu
from jax.experimental.pallas import tpu_sc as plsc

N = 10000
E = 320000
D = 128
LANES = 16
NCORES = 2
NSUB = 16
NW = NCORES * NSUB
EPW = E // NW
RB = 80
NB = EPW // RB
ROWS_PER_SUB = N // NSUB
ZROWS = 125


def _sc_compiler_params():
    cp = pltpu.CompilerParams()
    if "needs_layout_passes" in pltpu.CompilerParams.__dataclass_fields__:
        cp = dataclasses.replace(cp, needs_layout_passes=False)
    return cp


def _tc_prep_kernel(x_ref, w_ref, b_ref, wl_ref, wr_ref,
                    feat_ref, sa_ref, at_ref, m_ref):
    feat = lax.dot_general(x_ref[...], w_ref[...], (((1,), (1,)), ((), ())),
                           preferred_element_type=jnp.float32)
    feat = feat + b_ref[...]
    feat_ref[...] = feat
    sa = lax.dot_general(feat, wl_ref[...], (((1,), (0,)), ((), ())),
                         preferred_element_type=jnp.float32)
    at = lax.dot_general(feat, wr_ref[...], (((1,), (0,)), ((), ())),
                         preferred_element_type=jnp.float32)
    sa_ref[...] = sa
    at_ref[...] = at
    m = jnp.max(sa) + jnp.max(at)
    m_ref[...] = jnp.broadcast_to(jnp.where(m > 0, m, 0.01 * m), (1, LANES))


def _tc_recip_kernel(d_ref, r_ref):
    d = d_ref[0:1, :] + d_ref[1:2, :]
    r_ref[...] = 1.0 / jnp.maximum(d, 1e-30)


def _tc_final_kernel(acc_ref, h_ref, phis_ref):
    h = jnp.maximum(acc_ref[0] + acc_ref[1], 0.0)
    h_ref[...] = h
    phis_ref[...] = jnp.sum(h, axis=0, keepdims=True)


def _sc_edge_softmax(selfa, attn, m16, src3, dst3):
    mesh = plsc.VectorSubcoreMesh(core_axis_name="c", subcore_axis_name="s")

    @functools.partial(
        pl.kernel,
        out_type=(jax.ShapeDtypeStruct((NW, NB, RB), jnp.float32),
                  jax.ShapeDtypeStruct((NCORES, N), jnp.float32)),
        mesh=mesh,
        compiler_params=_sc_compiler_params(),
        scratch_types=[
            pltpu.VMEM((N,), jnp.float32),
            pltpu.VMEM((N,), jnp.float32),
            pltpu.VMEM((LANES,), jnp.float32),
            pltpu.VMEM((NB, RB), jnp.int32),
            pltpu.VMEM((NB, RB), jnp.int32),
            pltpu.VMEM((NB, RB), jnp.float32),
            pltpu.VMEM((2000,), jnp.float32),
            pltpu.SemaphoreType.DMA,
            pltpu.VMEM_SHARED((N,), jnp.float32),
        ],
    )
    def kern(selfa_hbm, attn_hbm, m_hbm, src_hbm, dst_hbm,
             ex_hbm, denomp_hbm,
             selfa_v, attn_v, m_v, src_v, dst_v, ex_v, zbuf, dsem, denom_sh):
        cid = lax.axis_index("c")
        sid = lax.axis_index("s")
        wid = sid * NCORES + cid

        pltpu.sync_copy(selfa_hbm, selfa_v)
        pltpu.sync_copy(attn_hbm, attn_v)
        pltpu.sync_copy(m_hbm, m_v)
        pltpu.sync_copy(src_hbm.at[wid], src_v)
        pltpu.sync_copy(dst_hbm.at[wid], dst_v)

        @pl.when(sid == 0)
        def _():
            @pl.loop(0, 2000 // LANES)
            def _(i):
                zbuf[pl.ds(i * LANES, LANES)] = jnp.zeros((LANES,), jnp.float32)

            @pl.loop(0, N // 2000)
            def _(t):
                pltpu.sync_copy(zbuf, denom_sh.at[pl.ds(t * 2000, 2000)])

        plsc.subcore_barrier()

        mvec = m_v[...]

        @plsc.parallel_loop(0, NB, unroll=2)
        def _(j):
            for k in range(RB // LANES):
                sl = pl.ds(k * LANES, LANES)
                a = (plsc.load_gather(selfa_v, [dst_v.at[j][sl]])
                     + plsc.load_gather(attn_v, [src_v.at[j][sl]]))
                a = jnp.where(a > 0, a, a * 0.01)
                ex_v.at[j][sl] = jnp.exp(a - mvec)

        @pl.loop(0, NB // 5)
        def _(g):
            hs = []
            for u in range(5):
                j = g * 5 + u
                hs.append(pltpu.async_copy(
                    ex_v.at[j], denom_sh.at[dst_v.at[j]], dsem, add=True))
            for h in hs:
                h.wait()

        pltpu.sync_copy(ex_v, ex_hbm.at[wid])
        plsc.subcore_barrier()

        @pl.when(sid == 0)
        def _():
            pltpu.sync_copy(denom_sh, denomp_hbm.at[cid])

    return kern(selfa, attn, m16, src3, dst3)


def _sc_aggregate(feat, ex3, recip, src3, dst3):
    mesh = plsc.VectorSubcoreMesh(core_axis_name="c", subcore_axis_name="s")

    CB = 8
    NCHUNK = NB // CB

    @functools.partial(
        pl.kernel,
        out_type=jax.ShapeDtypeStruct((NCORES, N, D), jnp.float32),
        mesh=mesh,
        compiler_params=_sc_compiler_params(),
        scratch_types=[
            pltpu.VMEM((N,), jnp.float32),
            pltpu.VMEM((CB, RB), jnp.int32),
            pltpu.VMEM((CB, RB), jnp.int32),
            pltpu.VMEM((CB, RB), jnp.float32),
            pltpu.VMEM((RB, D), jnp.float32),
            pltpu.VMEM((RB, D), jnp.float32),
            pltpu.SemaphoreType.DMA,
            pltpu.SemaphoreType.DMA,
            pltpu.VMEM_SHARED((N, D), jnp.float32),
        ],
    )
    def kern(feat_hbm, ex_hbm, recip_hbm, src_hbm, dst_hbm, acc_hbm,
             recip_v, src_c, dst_c, e_c, rows0_v, rows1_v, gsem, ssem,
             acc_sh):
        cid = lax.axis_index("c")
        sid = lax.axis_index("s")
        wid = sid * NCORES + cid

        pltpu.sync_copy(recip_hbm, recip_v)

        @pl.loop(0, RB)
        def _(r):
            for c in range(D // LANES):
                rows0_v.at[r][pl.ds(c * LANES, LANES)] = jnp.zeros(
                    (LANES,), jnp.float32)

        for t in range(7):
            pltpu.sync_copy(
                rows0_v, acc_sh.at[pl.ds(sid * ROWS_PER_SUB + t * RB, RB)])
        pltpu.sync_copy(rows0_v.at[pl.ds(0, 65)],
                        acc_sh.at[pl.ds(sid * ROWS_PER_SUB + 560, 65)])
        plsc.subcore_barrier()

        bufs = (rows0_v, rows1_v)

        def scale_block(j2, rows_v):
            for k in range(RB // LANES):
                sl = pl.ds(k * LANES, LANES)
                r16 = plsc.load_gather(recip_v, [dst_c.at[j2][sl]])
                e_c.at[j2][sl] = e_c.at[j2][sl] * r16

            @plsc.parallel_loop(0, RB, unroll=4)
            def _(r):
                esp = plsc.load_gather(
                    e_c, [jnp.full((LANES,), j2, jnp.int32),
                          jnp.full((LANES,), r, jnp.int32)])
                for c in range(D // LANES):
                    sl = pl.ds(c * LANES, LANES)
                    rows_v.at[r][sl] = rows_v.at[r][sl] * esp

        def do_chunk(base, nblk):
            pltpu.sync_copy(src_hbm.at[wid].at[pl.ds(base, nblk)],
                            src_c.at[pl.ds(0, nblk)])
            pltpu.sync_copy(dst_hbm.at[wid].at[pl.ds(base, nblk)],
                            dst_c.at[pl.ds(0, nblk)])
            pltpu.sync_copy(ex_hbm.at[wid].at[pl.ds(base, nblk)],
                            e_c.at[pl.ds(0, nblk)])

            h = pltpu.async_copy(feat_hbm.at[src_c.at[0]], bufs[0], gsem)
            sc = [None, None]
            for j2 in range(nblk):
                h.wait()
                if j2 + 1 < nblk:
                    if sc[(j2 + 1) % 2] is not None:
                        sc[(j2 + 1) % 2].wait()
                        sc[(j2 + 1) % 2] = None
                    h = pltpu.async_copy(feat_hbm.at[src_c.at[j2 + 1]],
                                         bufs[(j2 + 1) % 2], gsem)
                scale_block(j2, bufs[j2 % 2])
                sc[j2 % 2] = pltpu.async_copy(
                    bufs[j2 % 2], acc_sh.at[dst_c.at[j2]], ssem, add=True)
            for b in range(2):
                if sc[b] is not None:
                    sc[b].wait()

        @pl.loop(0, NCHUNK)
        def _(jj):
            do_chunk(jj * CB, CB)

        do_chunk(NCHUNK * CB, NB - NCHUNK * CB)

        plsc.subcore_barrier()
        pltpu.sync_copy(acc_sh.at[pl.ds(sid * 624, 624)],
                        acc_hbm.at[cid].at[pl.ds(sid * 624, 624)])

        @pl.when(sid == 0)
        def _():
            pltpu.sync_copy(acc_sh.at[pl.ds(9984, 16)],
                            acc_hbm.at[cid].at[pl.ds(9984, 16)])

    return kern(feat, ex3, recip, src3, dst3)


def kernel(x, edge_index, W_fc, b_fc, w_l, w_r):
    feat, sa, at, m11 = pl.pallas_call(
        _tc_prep_kernel,
        out_shape=[
            jax.ShapeDtypeStruct((N, D), jnp.float32),
            jax.ShapeDtypeStruct((N, 1), jnp.float32),
            jax.ShapeDtypeStruct((N, 1), jnp.float32),
            jax.ShapeDtypeStruct((1, LANES), jnp.float32),
        ],
    )(x, W_fc, b_fc.reshape(1, D), w_l.reshape(D, 1), w_r.reshape(D, 1))

    m16 = m11.reshape(LANES)
    src3 = edge_index[0].reshape(NW, NB, RB)
    dst3 = edge_index[1].reshape(NW, NB, RB)

    ex3, denomp = _sc_edge_softmax(sa.reshape(N), at.reshape(N), m16,
                                   src3, dst3)
    recip = pl.pallas_call(
        _tc_recip_kernel,
        out_shape=jax.ShapeDtypeStruct((1, N), jnp.float32),
    )(denomp)
    accp = _sc_aggregate(feat, ex3, recip.reshape(N), src3, dst3)

    h, phis = pl.pallas_call(
        _tc_final_kernel,
        out_shape=[
            jax.ShapeDtypeStruct((N, D), jnp.float32),
            jax.ShapeDtypeStruct((1, D), jnp.float32),
        ],
    )(accp)
    return h, phis.reshape(D)


# speedup vs baseline: 1.1261x; 1.0785x over previous
import dataclasses
import functools

import jax
import jax.numpy as jnp
from jax import lax
from jax.experimental import pallas as pl
from jax.experimental.pallas import tpu as pltpu
from jax.experimental.pallas import tpu_sc as plsc

N = 10000
E = 320000
D = 128
LANES = 16
NCORES = 2
NSUB = 16
NW = NCORES * NSUB
EPW = E // NW
RB = 80
NB = EPW // RB
ROWS_PER_SUB = N // NSUB
ZROWS = 125


def _sc_compiler_params():
    cp = pltpu.CompilerParams()
    if "needs_layout_passes" in pltpu.CompilerParams.__dataclass_fields__:
        cp = dataclasses.replace(cp, needs_layout_passes=False)
    return cp


def _tc_prep_kernel(x_ref, w_ref, b_ref, wl_ref, wr_ref,
                    feat_ref, sa_ref, at_ref, m_ref):
    feat = lax.dot_general(x_ref[...], w_ref[...], (((1,), (1,)), ((), ())),
                           preferred_element_type=jnp.float32)
    feat = feat + b_ref[...]
    feat_ref[...] = feat
    sa = lax.dot_general(feat, wl_ref[...], (((1,), (0,)), ((), ())),
                         preferred_element_type=jnp.float32)
    at = lax.dot_general(feat, wr_ref[...], (((1,), (0,)), ((), ())),
                         preferred_element_type=jnp.float32)
    sa_ref[...] = sa
    at_ref[...] = at
    m = jnp.max(sa) + jnp.max(at)
    m_ref[...] = jnp.broadcast_to(jnp.where(m > 0, m, 0.01 * m), (1, LANES))


def _tc_recip_kernel(d_ref, r_ref):
    d = d_ref[0:1, :] + d_ref[1:2, :]
    r_ref[...] = 1.0 / jnp.maximum(d, 1e-30)


def _tc_final_kernel(acc_ref, h_ref, phis_ref):
    h = jnp.maximum(acc_ref[0] + acc_ref[1], 0.0)
    h_ref[...] = h
    phis_ref[...] = jnp.sum(h, axis=0, keepdims=True)


def _sc_edge_softmax(selfa, attn, m16, src3, dst3):
    mesh = plsc.VectorSubcoreMesh(core_axis_name="c", subcore_axis_name="s")

    @functools.partial(
        pl.kernel,
        out_type=(jax.ShapeDtypeStruct((NW, NB, RB), jnp.float32),
                  jax.ShapeDtypeStruct((NCORES, N), jnp.float32)),
        mesh=mesh,
        compiler_params=_sc_compiler_params(),
        scratch_types=[
            pltpu.VMEM((N,), jnp.float32),
            pltpu.VMEM((N,), jnp.float32),
            pltpu.VMEM((LANES,), jnp.float32),
            pltpu.VMEM((NB, RB), jnp.int32),
            pltpu.VMEM((NB, RB), jnp.int32),
            pltpu.VMEM((NB, RB), jnp.float32),
            pltpu.VMEM((2000,), jnp.float32),
            pltpu.SemaphoreType.DMA,
            pltpu.VMEM_SHARED((N,), jnp.float32),
        ],
    )
    def kern(selfa_hbm, attn_hbm, m_hbm, src_hbm, dst_hbm,
             ex_hbm, denomp_hbm,
             selfa_v, attn_v, m_v, src_v, dst_v, ex_v, zbuf, dsem, denom_sh):
        cid = lax.axis_index("c")
        sid = lax.axis_index("s")
        wid = sid * NCORES + cid

        hs = [pltpu.async_copy(selfa_hbm, selfa_v, dsem),
              pltpu.async_copy(attn_hbm, attn_v, dsem),
              pltpu.async_copy(m_hbm, m_v, dsem),
              pltpu.async_copy(src_hbm.at[wid], src_v, dsem),
              pltpu.async_copy(dst_hbm.at[wid], dst_v, dsem)]
        for h in hs:
            h.wait()

        @pl.when(sid == 0)
        def _():
            @pl.loop(0, 2000 // LANES)
            def _(i):
                zbuf[pl.ds(i * LANES, LANES)] = jnp.zeros((LANES,), jnp.float32)

            @pl.loop(0, N // 2000)
            def _(t):
                pltpu.sync_copy(zbuf, denom_sh.at[pl.ds(t * 2000, 2000)])

        plsc.subcore_barrier()

        mvec = m_v[...]

        @plsc.parallel_loop(0, NB, unroll=2)
        def _(j):
            for k in range(RB // LANES):
                sl = pl.ds(k * LANES, LANES)
                a = (plsc.load_gather(selfa_v, [dst_v.at[j][sl]])
                     + plsc.load_gather(attn_v, [src_v.at[j][sl]]))
                a = jnp.where(a > 0, a, a * 0.01)
                ex_v.at[j][sl] = jnp.exp(a - mvec)

        @pl.loop(0, NB // 5)
        def _(g):
            hs = []
            for u in range(5):
                j = g * 5 + u
                hs.append(pltpu.async_copy(
                    ex_v.at[j], denom_sh.at[dst_v.at[j]], dsem, add=True))
            for h in hs:
                h.wait()

        pltpu.sync_copy(ex_v, ex_hbm.at[wid])
        plsc.subcore_barrier()

        @pl.when(sid == 0)
        def _():
            pltpu.sync_copy(denom_sh, denomp_hbm.at[cid])

    return kern(selfa, attn, m16, src3, dst3)


def _sc_aggregate(feat, ex3, recip, src3, dst3):
    mesh = plsc.VectorSubcoreMesh(core_axis_name="c", subcore_axis_name="s")

    CB = 8
    NCHUNK = NB // CB

    @functools.partial(
        pl.kernel,
        out_type=jax.ShapeDtypeStruct((NCORES, N, D), jnp.float32),
        mesh=mesh,
        compiler_params=_sc_compiler_params(),
        scratch_types=[
            pltpu.VMEM((N,), jnp.float32),
            pltpu.VMEM((CB, RB), jnp.int32),
            pltpu.VMEM((CB, RB), jnp.int32),
            pltpu.VMEM((CB, RB), jnp.float32),
            pltpu.VMEM((RB, D), jnp.float32),
            pltpu.VMEM((RB, D), jnp.float32),
            pltpu.SemaphoreType.DMA,
            pltpu.SemaphoreType.DMA,
            pltpu.VMEM_SHARED((N, D), jnp.float32),
        ],
    )
    def kern(feat_hbm, ex_hbm, recip_hbm, src_hbm, dst_hbm, acc_hbm,
             recip_v, src_c, dst_c, e_c, rows0_v, rows1_v, gsem, ssem,
             acc_sh):
        cid = lax.axis_index("c")
        sid = lax.axis_index("s")
        wid = sid * NCORES + cid

        hr = pltpu.async_copy(recip_hbm, recip_v, gsem)

        @pl.loop(0, RB)
        def _(r):
            for c in range(D // LANES):
                rows0_v.at[r][pl.ds(c * LANES, LANES)] = jnp.zeros(
                    (LANES,), jnp.float32)

        for t in range(7):
            pltpu.sync_copy(
                rows0_v, acc_sh.at[pl.ds(sid * ROWS_PER_SUB + t * RB, RB)])
        pltpu.sync_copy(rows0_v.at[pl.ds(0, 65)],
                        acc_sh.at[pl.ds(sid * ROWS_PER_SUB + 560, 65)])
        hr.wait()
        plsc.subcore_barrier()

        bufs = (rows0_v, rows1_v)

        def scale_block(j2, rows_v):
            for k in range(RB // LANES):
                sl = pl.ds(k * LANES, LANES)
                r16 = plsc.load_gather(recip_v, [dst_c.at[j2][sl]])
                e_c.at[j2][sl] = e_c.at[j2][sl] * r16

            @plsc.parallel_loop(0, RB, unroll=4)
            def _(r):
                esp = plsc.load_gather(
                    e_c, [jnp.full((LANES,), j2, jnp.int32),
                          jnp.full((LANES,), r, jnp.int32)])
                for c in range(D // LANES):
                    sl = pl.ds(c * LANES, LANES)
                    rows_v.at[r][sl] = rows_v.at[r][sl] * esp

        def do_chunk(base, nblk):
            hs = [pltpu.async_copy(src_hbm.at[wid].at[pl.ds(base, nblk)],
                                   src_c.at[pl.ds(0, nblk)], gsem),
                  pltpu.async_copy(dst_hbm.at[wid].at[pl.ds(base, nblk)],
                                   dst_c.at[pl.ds(0, nblk)], gsem),
                  pltpu.async_copy(ex_hbm.at[wid].at[pl.ds(base, nblk)],
                                   e_c.at[pl.ds(0, nblk)], gsem)]
            for hh in hs:
                hh.wait()

            h = pltpu.async_copy(feat_hbm.at[src_c.at[0]], bufs[0], gsem)
            sc = [None, None]
            for j2 in range(nblk):
                h.wait()
                if j2 + 1 < nblk:
                    if sc[(j2 + 1) % 2] is not None:
                        sc[(j2 + 1) % 2].wait()
                        sc[(j2 + 1) % 2] = None
                    h = pltpu.async_copy(feat_hbm.at[src_c.at[j2 + 1]],
                                         bufs[(j2 + 1) % 2], gsem)
                scale_block(j2, bufs[j2 % 2])
                sc[j2 % 2] = pltpu.async_copy(
                    bufs[j2 % 2], acc_sh.at[dst_c.at[j2]], ssem, add=True)
            for b in range(2):
                if sc[b] is not None:
                    sc[b].wait()

        @pl.loop(0, NCHUNK)
        def _(jj):
            do_chunk(jj * CB, CB)

        do_chunk(NCHUNK * CB, NB - NCHUNK * CB)

        plsc.subcore_barrier()
        pltpu.sync_copy(acc_sh.at[pl.ds(sid * 624, 624)],
                        acc_hbm.at[cid].at[pl.ds(sid * 624, 624)])

        @pl.when(sid == 0)
        def _():
            pltpu.sync_copy(acc_sh.at[pl.ds(9984, 16)],
                            acc_hbm.at[cid].at[pl.ds(9984, 16)])

    return kern(feat, ex3, recip, src3, dst3)


def kernel(x, edge_index, W_fc, b_fc, w_l, w_r):
    feat, sa, at, m11 = pl.pallas_call(
        _tc_prep_kernel,
        out_shape=[
            jax.ShapeDtypeStruct((N, D), jnp.float32),
            jax.ShapeDtypeStruct((N, 1), jnp.float32),
            jax.ShapeDtypeStruct((N, 1), jnp.float32),
            jax.ShapeDtypeStruct((1, LANES), jnp.float32),
        ],
    )(x, W_fc, b_fc.reshape(1, D), w_l.reshape(D, 1), w_r.reshape(D, 1))

    m16 = m11.reshape(LANES)
    src3 = edge_index[0].reshape(NW, NB, RB)
    dst3 = edge_index[1].reshape(NW, NB, RB)

    ex3, denomp = _sc_edge_softmax(sa.reshape(N), at.reshape(N), m16,
                                   src3, dst3)
    recip = pl.pallas_call(
        _tc_recip_kernel,
        out_shape=jax.ShapeDtypeStruct((1, N), jnp.float32),
    )(denomp)
    accp = _sc_aggregate(feat, ex3, recip.reshape(N), src3, dst3)

    h, phis = pl.pallas_call(
        _tc_final_kernel,
        out_shape=[
            jax.ShapeDtypeStruct((N, D), jnp.float32),
            jax.ShapeDtypeStruct((1, D), jnp.float32),
        ],
    )(accp)
    return h, phis.reshape(D)
